# Initial kernel scaffold; baseline (speedup 1.0000x reference)
import jax
import jax.numpy as jnp
from jax.experimental import pallas as pl


def kernel(detection_scores, detection_classes, detection_boxes, detection_masks, segmentation_outputs):
    raise NotImplementedError("write your pallas kernel here")


# speedup vs baseline: 2327.1236x; 2327.1236x over previous
import functools

import numpy as np
import jax
import jax.numpy as jnp
from jax.experimental import pallas as pl
from jax.experimental.pallas import tpu as pltpu

OUT_H = 512
OUT_W = 512
SRC_HW = 128
MH = 28
MW = 28
NPAD = 128
STUFF_OFFSET = 90.0
MASK_THR = 0.5
SCORE_THR = 0.05
VOID_ENC = 6553600.0


def _interp_matrix(out_size: int, in_size: int) -> np.ndarray:
    i = np.arange(out_size, dtype=np.float32)
    src = (i + 0.5) * (in_size / out_size) - 0.5
    j = np.arange(in_size, dtype=np.float32)
    w = np.maximum(0.0, 1.0 - np.abs(j[None, :] - src[:, None])).astype(np.float32)
    w = w / np.sum(w, axis=1, keepdims=True)
    return w.astype(np.float32)


def _panoptic_kernel(nreal, nchan, sa_ref, sb_ref, cls_ref, box_ref, m_ref,
                     seg_ref, wy_ref, wxt_ref, cat_ref, inst_ref, encv_ref):
    f32 = jnp.float32

    si = sa_ref[0]
    sj = sb_ref[0]
    ii = jax.lax.broadcasted_iota(jnp.int32, (NPAD, NPAD), 0)
    jj = jax.lax.broadcasted_iota(jnp.int32, (NPAD, NPAD), 1)
    beats = (sj > si) | ((sj == si) & (jj < ii))
    ranks = jnp.sum(beats.astype(f32), axis=1, keepdims=True)
    clsv = cls_ref[0]
    dv = jax.lax.broadcasted_iota(jnp.int32, (NPAD, 1), 0).astype(f32)
    validv = si > SCORE_THR
    encv = jnp.where(validv, ranks * 65536.0 + clsv * 128.0 + dv, VOID_ENC)
    encv_ref[...] = encv

    wy = wy_ref[...]
    wxt = wxt_ref[...]

    def ch_body(c, carry):
        best, bidx = carry
        x = seg_ref[0, c]
        t = jnp.dot(wy, x, preferred_element_type=f32,
                    precision=jax.lax.Precision.HIGHEST)
        v = jnp.dot(t, wxt, preferred_element_type=f32,
                    precision=jax.lax.Precision.HIGHEST)
        upd = v > best
        return jnp.where(upd, v, best), jnp.where(upd, c.astype(f32), bidx)

    best0 = jnp.full((OUT_H, OUT_W), -jnp.inf, f32)
    bidx0 = jnp.zeros((OUT_H, OUT_W), f32)
    _, segf = jax.lax.fori_loop(0, nchan, ch_body, (best0, bidx0))

    yi = jax.lax.broadcasted_iota(jnp.int32, (OUT_H, MH), 0).astype(f32)
    ky = jax.lax.broadcasted_iota(jnp.int32, (OUT_H, MH), 1).astype(f32)
    jx = jax.lax.broadcasted_iota(jnp.int32, (MW, OUT_W), 0).astype(f32)
    xi = jax.lax.broadcasted_iota(jnp.int32, (MW, OUT_W), 1).astype(f32)

    def det_body(d, encmin):
        bx = box_ref[0, pl.ds(d, 1), :]
        ymin = bx[:, 0:1]
        xmin = bx[:, 1:2]
        ymaxc = jnp.clip(bx[:, 2:3] + 1.0, 0.0, float(OUT_H))
        xmaxc = jnp.clip(bx[:, 3:4] + 1.0, 0.0, float(OUT_W))
        bh = jnp.maximum(ymaxc - ymin, 1.0)
        bw = jnp.maximum(xmaxc - xmin, 1.0)
        fy = ((yi - ymin) + 0.5) * (MH / bh)
        sy = jnp.clip(jnp.floor(fy), 0.0, float(MH - 1))
        oy = ((ky == sy) & (yi >= ymin) & (yi < ymaxc)).astype(f32)
        fx = ((xi - xmin) + 0.5) * (MW / bw)
        sx = jnp.clip(jnp.floor(fx), 0.0, float(MW - 1))
        oxt = ((jx == sx) & (xi >= xmin) & (xi < xmaxc)).astype(f32)
        bm = (m_ref[0, d] > MASK_THR).astype(f32)
        q = jnp.dot(oy, bm, preferred_element_type=f32)
        cov = jnp.dot(q, oxt, preferred_element_type=f32)
        e = encv_ref[pl.ds(d, 1), :]
        return jnp.minimum(encmin, jnp.where(cov > 0.5, e, VOID_ENC))

    enc0 = jnp.full((OUT_H, OUT_W), VOID_ENC, f32)
    encmin = jax.lax.fori_loop(0, nreal, det_body, enc0)

    found = encmin < VOID_ENC
    r = jnp.floor(encmin * (1.0 / 65536.0))
    rem = encmin - r * 65536.0
    cls = jnp.floor(rem * (1.0 / 128.0))
    dd = rem - cls * 128.0
    catf = jnp.where(found, cls, 0.0)
    instf = jnp.where(found, dd + 1.0, -1.0)
    stuff = (segf != 0.0) & (segf != 1.0)
    catf = jnp.where((~found) & stuff, segf + STUFF_OFFSET, catf)
    cat_ref[0] = catf.astype(jnp.int32)
    inst_ref[0] = instf.astype(jnp.int32)


def _run(detection_scores, detection_classes, detection_boxes,
         detection_masks, segmentation_outputs, interpret):
    B, N = detection_scores.shape
    C = segmentation_outputs.shape[-1]

    pad = NPAD - N
    scores = jnp.pad(detection_scores, ((0, 0), (0, pad)),
                     constant_values=-1.0)
    classes = jnp.pad(detection_classes, ((0, 0), (0, pad)))
    boxes = jnp.pad(detection_boxes, ((0, 0), (0, pad), (0, 0)))
    masks = jnp.pad(detection_masks, ((0, 0), (0, pad), (0, 0), (0, 0)))

    sa = scores.reshape(B, NPAD, 1)
    sb = scores.reshape(B, 1, NPAD)
    cls_a = classes.reshape(B, NPAD, 1)
    seg_t = jnp.transpose(segmentation_outputs, (0, 3, 1, 2))

    wy = jnp.asarray(_interp_matrix(OUT_H, SRC_HW))
    wxt = jnp.asarray(_interp_matrix(OUT_W, SRC_HW).T)

    grid = (B,)
    kern = functools.partial(_panoptic_kernel, N, C)
    cat, inst = pl.pallas_call(
        kern,
        grid=grid,
        in_specs=[
            pl.BlockSpec((1, NPAD, 1), lambda b: (b, 0, 0)),
            pl.BlockSpec((1, 1, NPAD), lambda b: (b, 0, 0)),
            pl.BlockSpec((1, NPAD, 1), lambda b: (b, 0, 0)),
            pl.BlockSpec((1, NPAD, 4), lambda b: (b, 0, 0)),
            pl.BlockSpec((1, NPAD, MH, MW), lambda b: (b, 0, 0, 0)),
            pl.BlockSpec((1, C, SRC_HW, SRC_HW), lambda b: (b, 0, 0, 0)),
            pl.BlockSpec((OUT_H, SRC_HW), lambda b: (0, 0)),
            pl.BlockSpec((SRC_HW, OUT_W), lambda b: (0, 0)),
        ],
        out_specs=[
            pl.BlockSpec((1, OUT_H, OUT_W), lambda b: (b, 0, 0)),
            pl.BlockSpec((1, OUT_H, OUT_W), lambda b: (b, 0, 0)),
        ],
        out_shape=[
            jax.ShapeDtypeStruct((B, OUT_H, OUT_W), jnp.int32),
            jax.ShapeDtypeStruct((B, OUT_H, OUT_W), jnp.int32),
        ],
        scratch_shapes=[pltpu.VMEM((NPAD, 1), jnp.float32)],
        interpret=interpret,
    )(sa, sb, cls_a, boxes, masks, seg_t, wy, wxt)
    return cat, inst


def kernel(detection_scores, detection_classes, detection_boxes,
           detection_masks, segmentation_outputs):
    return _run(detection_scores, detection_classes, detection_boxes,
                detection_masks, segmentation_outputs, False)
